# baseline (speedup 1.0000x reference)
import jax
import jax.numpy as jnp
from jax import lax
from jax.experimental import pallas as pl
from jax.experimental.pallas import tpu as pltpu
from jax.experimental.pallas import tpu_sc as plsc

N = 10000
E = 320000
D = 128
G = 64

NC = 2
NS = 16
HD = D // NC
C = 125
EPT = E // NS
KCH = EPT // C
NBUF = 3
RPT = N // NS
ZR = 25
WPT = 632
WLAST = N - (NS - 1) * WPT

BLK = 400
NBLK = N // BLK


def _spmm_body(x2_hbm, src_hbm, dst_hbm, w_hbm, out_hbm,
               acc, sidx, didx, wbuf, ring, zbuf, gsem, ssem):
    cid = lax.axis_index("c")
    sid = lax.axis_index("s")
    xh = x2_hbm.at[cid]

    zero16 = jnp.zeros((16,), jnp.float32)
    for r in range(ZR):
        for j in range(HD // 16):
            zbuf[r, pl.ds(16 * j, 16)] = zero16
    for k in range(RPT // ZR):
        pltpu.sync_copy(zbuf, acc.at[pl.ds(sid * RPT + k * ZR, ZR)])

    pltpu.sync_copy(src_hbm.at[pl.ds(sid * KCH, KCH)], sidx)
    pltpu.sync_copy(dst_hbm.at[pl.ds(sid * KCH, KCH)], didx)
    pltpu.sync_copy(w_hbm.at[pl.ds(sid * KCH, KCH)], wbuf)
    plsc.subcore_barrier()

    pltpu.async_copy(xh.at[sidx.at[0]], ring.at[0], gsem)
    pltpu.async_copy(xh.at[sidx.at[1]], ring.at[1], gsem)

    def chunk(k, carry):
        b = lax.rem(k, NBUF)
        pltpu.make_async_copy(xh.at[sidx.at[k]], ring.at[b], gsem).wait()

        def scale(i, c2):
            ws = plsc.load_gather(wbuf, [jnp.full((16,), k, jnp.int32),
                                         jnp.full((16,), i, jnp.int32)])
            for j in range(HD // 16):
                sl = pl.ds(16 * j, 16)
                ring[b, i, sl] = ring[b, i, sl] * ws
            return c2
        lax.fori_loop(0, C, scale, 0, unroll=4)

        pltpu.async_copy(ring.at[b], acc.at[didx.at[k]], ssem, add=True)

        @pl.when(k >= 1)
        def _drain():
            pb = lax.rem(k - 1, NBUF)
            pltpu.make_async_copy(ring.at[pb], acc.at[didx.at[k - 1]],
                                  ssem).wait()

        @pl.when(k + 2 < KCH)
        def _prefetch():
            nb = lax.rem(k + 2, NBUF)
            pltpu.async_copy(xh.at[sidx.at[k + 2]], ring.at[nb], gsem)
        return carry
    lax.fori_loop(0, KCH, chunk, 0)
    pltpu.make_async_copy(ring.at[lax.rem(KCH - 1, NBUF)],
                          acc.at[didx.at[KCH - 1]], ssem).wait()

    plsc.subcore_barrier()

    @pl.when(sid < NS - 1)
    def _wmain():
        pltpu.sync_copy(acc.at[pl.ds(sid * WPT, WPT)],
                        out_hbm.at[cid, pl.ds(sid * WPT, WPT)])

    @pl.when(sid == NS - 1)
    def _wtail():
        pltpu.sync_copy(acc.at[pl.ds((NS - 1) * WPT, WLAST)],
                        out_hbm.at[cid, pl.ds((NS - 1) * WPT, WLAST)])


def _spmm_sc(x2, src2, dst2, w2):
    mesh = plsc.VectorSubcoreMesh(core_axis_name="c", subcore_axis_name="s",
                                  num_cores=NC, num_subcores=NS)
    f = pl.kernel(
        _spmm_body,
        out_type=jax.ShapeDtypeStruct((NC, N, HD), jnp.float32),
        mesh=mesh,
        compiler_params=pltpu.CompilerParams(use_tc_tiling_on_sc=False,
                                             needs_layout_passes=False),
        scratch_types=[
            pltpu.VMEM_SHARED((N, HD), jnp.float32),
            pltpu.VMEM((KCH, C), jnp.int32),
            pltpu.VMEM((KCH, C), jnp.int32),
            pltpu.VMEM((KCH, C), jnp.float32),
            pltpu.VMEM((NBUF, C, HD), jnp.float32),
            pltpu.VMEM((ZR, HD), jnp.float32),
            pltpu.SemaphoreType.DMA,
            pltpu.SemaphoreType.DMA,
        ],
    )
    return f(x2, src2, dst2, w2)


def _dotT(a, b):
    return lax.dot_general(a, b, (((1,), (1,)), ((), ())),
                           preferred_element_type=jnp.float32)


def _dense_body(ab, xb, wr, wt, br, out):
    s = jnp.concatenate([ab[0], ab[1]], axis=1)
    h = _dotT(s, wr[...]) + _dotT(xb[...], wt[...]) + br[...]
    h = jnp.maximum(h, 0.0)
    out[0] = h[:, :HD]
    out[1] = h[:, HD:]


def _dense_tc(agg, x, W_rel, b_rel, W_root):
    b2d = b_rel.reshape(1, D)
    grid = (NBLK,)
    return pl.pallas_call(
        _dense_body,
        grid=grid,
        in_specs=[
            pl.BlockSpec((NC, BLK, HD), lambda k: (0, k, 0)),
            pl.BlockSpec((BLK, D), lambda k: (k, 0)),
            pl.BlockSpec((D, D), lambda k: (0, 0)),
            pl.BlockSpec((D, D), lambda k: (0, 0)),
            pl.BlockSpec((1, D), lambda k: (0, 0)),
        ],
        out_specs=pl.BlockSpec((NC, BLK, HD), lambda k: (0, k, 0)),
        out_shape=jax.ShapeDtypeStruct((NC, N, HD), jnp.float32),
    )(agg, x, W_rel, W_root, b2d)


def _head_body(ab, hb, wr, wt, br, batchb, wl1, bl1, wl2, bl2,
               out, psum, cnt):
    k = pl.program_id(0)

    @pl.when(k == 0)
    def _init():
        psum[...] = jnp.zeros_like(psum)
        cnt[...] = jnp.zeros_like(cnt)

    s = jnp.concatenate([ab[0], ab[1]], axis=1)
    hin = jnp.concatenate([hb[0], hb[1]], axis=1)
    h = _dotT(s, wr[...]) + _dotT(hin, wt[...]) + br[...]
    h = jnp.maximum(h, 0.0)
    bvec = batchb[0, 0, :]
    onehot = (lax.broadcasted_iota(jnp.int32, (G, BLK), 0)
              == bvec[None, :]).astype(jnp.float32)
    psum[...] += jnp.dot(onehot, h, preferred_element_type=jnp.float32)
    cnt[...] += jnp.broadcast_to(
        jnp.sum(onehot, axis=1, keepdims=True), (G, D))

    @pl.when(k == NBLK - 1)
    def _fin():
        p = psum[...] / jnp.maximum(cnt[...], 1.0)
        z = jnp.maximum(_dotT(p, wl1[...]) + bl1[...], 0.0)
        out[...] = _dotT(z, wl2[...]) + bl2[...]


def _head_tc(agg, h1, W_rel, b_rel, W_root, batchr, W_lin1, b_lin1,
             W_lin2p, b_lin2p):
    b2d = b_rel.reshape(1, D)
    grid = (NBLK,)
    return pl.pallas_call(
        _head_body,
        grid=grid,
        in_specs=[
            pl.BlockSpec((NC, BLK, HD), lambda k: (0, k, 0)),
            pl.BlockSpec((NC, BLK, HD), lambda k: (0, k, 0)),
            pl.BlockSpec((D, D), lambda k: (0, 0)),
            pl.BlockSpec((D, D), lambda k: (0, 0)),
            pl.BlockSpec((1, D), lambda k: (0, 0)),
            pl.BlockSpec((1, 1, BLK), lambda k: (k, 0, 0)),
            pl.BlockSpec((16, D), lambda k: (0, 0)),
            pl.BlockSpec((1, 16), lambda k: (0, 0)),
            pl.BlockSpec((16, 16), lambda k: (0, 0)),
            pl.BlockSpec((1, 16), lambda k: (0, 0)),
        ],
        out_specs=pl.BlockSpec((G, 16), lambda k: (0, 0)),
        out_shape=jax.ShapeDtypeStruct((G, 16), jnp.float32),
        scratch_shapes=[
            pltpu.VMEM((G, D), jnp.float32),
            pltpu.VMEM((G, D), jnp.float32),
        ],
    )(agg, h1, W_rel, W_root, b2d, batchr,
      W_lin1, b_lin1.reshape(1, 16), W_lin2p, b_lin2p)


def kernel(x, edge_index, edge_attr, batch, W1_rel, b1_rel, W1_root,
           W2_rel, b2_rel, W2_root, W_lin1, b_lin1, W_lin2, b_lin2):
    src2 = edge_index[0].reshape(E // C, C)
    dst2 = edge_index[1].reshape(E // C, C)
    w2 = edge_attr.reshape(E // C, C)
    batchr = batch.reshape(NBLK, 1, BLK)
    W_lin2p = jnp.zeros((16, 16), jnp.float32).at[0].set(W_lin2[0])
    b_lin2p = jnp.zeros((1, 16), jnp.float32).at[0, 0].set(b_lin2[0])
    xsplit = x.reshape(N, NC, HD).swapaxes(0, 1)

    agg1 = _spmm_sc(xsplit, src2, dst2, w2)
    h1 = _dense_tc(agg1, x, W1_rel, b1_rel, W1_root)
    agg2 = _spmm_sc(h1, src2, dst2, w2)
    out16 = _head_tc(agg2, h1, W2_rel, b2_rel, W2_root, batchr,
                     W_lin1, b_lin1, W_lin2p, b_lin2p)
    return out16[:, 0:1]


# speedup vs baseline: 2.4586x; 2.4586x over previous
import jax
import jax.numpy as jnp
from jax import lax
from jax.experimental import pallas as pl
from jax.experimental.pallas import tpu as pltpu
from jax.experimental.pallas import tpu_sc as plsc

N = 10000
E = 320000
D = 128
G = 64

NC = 2
NS = 16
HD = D // NC
C = 100
EPT = E // NS
KCH = EPT // C
NBUF = 3
RPT = N // NS
ZR = 25
WPT = 632
WLAST = N - (NS - 1) * WPT

BLK = 400
NBLK = N // BLK


_SPLAT_DNUMS = lax.GatherDimensionNumbers(
    offset_dims=(), collapsed_slice_dims=(0,), start_index_map=(0,))


def _splat(vec16, lane):
    idx = jnp.full((16, 1), lane, jnp.int32)
    return lax.gather(vec16, idx, _SPLAT_DNUMS, (1,),
                      mode=lax.GatherScatterMode.PROMISE_IN_BOUNDS)


def _spmm_body(x2_hbm, src_hbm, dst_hbm, w_hbm, out_hbm,
               acc, sidx, didx, wbuf, ring, zbuf, gsem, ssem):
    cid = lax.axis_index("c")
    sid = lax.axis_index("s")
    xh = x2_hbm.at[cid]

    zero16 = jnp.zeros((16,), jnp.float32)
    for r in range(ZR):
        for j in range(HD // 16):
            zbuf[r, pl.ds(16 * j, 16)] = zero16
    for k in range(RPT // ZR):
        pltpu.sync_copy(zbuf, acc.at[pl.ds(sid * RPT + k * ZR, ZR)])

    pltpu.sync_copy(src_hbm.at[pl.ds(sid * KCH, KCH)], sidx)
    pltpu.sync_copy(dst_hbm.at[pl.ds(sid * KCH, KCH)], didx)
    pltpu.sync_copy(w_hbm.at[pl.ds(sid * KCH, KCH)], wbuf)
    plsc.subcore_barrier()

    pltpu.async_copy(xh.at[sidx.at[0]], ring.at[0], gsem)
    pltpu.async_copy(xh.at[sidx.at[1]], ring.at[1], gsem)

    def chunk(k, carry):
        b = lax.rem(k, NBUF)
        pltpu.make_async_copy(xh.at[sidx.at[k]], ring.at[b], gsem).wait()

        def scale_run(goff, lo):
            wv = wbuf[k, pl.ds(goff, 16)]
            for i16 in range(lo, 16):
                ws = _splat(wv, i16)
                e = goff + i16
                for j in range(HD // 16):
                    sl = pl.ds(16 * j, 16)
                    ring[b, e, sl] = ring[b, e, sl] * ws
        for g in range(C // 16):
            scale_run(g * 16, 0)
        if C % 16:
            scale_run(C - 16, 16 - C % 16)

        pltpu.async_copy(ring.at[b], acc.at[didx.at[k]], ssem, add=True)

        @pl.when(k >= 1)
        def _drain():
            pb = lax.rem(k - 1, NBUF)
            pltpu.make_async_copy(ring.at[pb], acc.at[didx.at[k - 1]],
                                  ssem).wait()

        @pl.when(k + 2 < KCH)
        def _prefetch():
            nb = lax.rem(k + 2, NBUF)
            pltpu.async_copy(xh.at[sidx.at[k + 2]], ring.at[nb], gsem)
        return carry
    lax.fori_loop(0, KCH, chunk, 0)
    pltpu.make_async_copy(ring.at[lax.rem(KCH - 1, NBUF)],
                          acc.at[didx.at[KCH - 1]], ssem).wait()

    plsc.subcore_barrier()

    @pl.when(sid < NS - 1)
    def _wmain():
        pltpu.sync_copy(acc.at[pl.ds(sid * WPT, WPT)],
                        out_hbm.at[cid, pl.ds(sid * WPT, WPT)])

    @pl.when(sid == NS - 1)
    def _wtail():
        pltpu.sync_copy(acc.at[pl.ds((NS - 1) * WPT, WLAST)],
                        out_hbm.at[cid, pl.ds((NS - 1) * WPT, WLAST)])


def _spmm_sc(x2, src2, dst2, w2):
    mesh = plsc.VectorSubcoreMesh(core_axis_name="c", subcore_axis_name="s",
                                  num_cores=NC, num_subcores=NS)
    f = pl.kernel(
        _spmm_body,
        out_type=jax.ShapeDtypeStruct((NC, N, HD), jnp.float32),
        mesh=mesh,
        compiler_params=pltpu.CompilerParams(use_tc_tiling_on_sc=False,
                                             needs_layout_passes=False),
        scratch_types=[
            pltpu.VMEM_SHARED((N, HD), jnp.float32),
            pltpu.VMEM((KCH, C), jnp.int32),
            pltpu.VMEM((KCH, C), jnp.int32),
            pltpu.VMEM((KCH, C), jnp.float32),
            pltpu.VMEM((NBUF, C, HD), jnp.float32),
            pltpu.VMEM((ZR, HD), jnp.float32),
            pltpu.SemaphoreType.DMA,
            pltpu.SemaphoreType.DMA,
        ],
    )
    return f(x2, src2, dst2, w2)


def _dotT(a, b):
    return lax.dot_general(a, b, (((1,), (1,)), ((), ())),
                           preferred_element_type=jnp.float32)


def _dense_body(ab, xb, wr, wt, br, out):
    s = jnp.concatenate([ab[0], ab[1]], axis=1)
    h = _dotT(s, wr[...]) + _dotT(xb[...], wt[...]) + br[...]
    h = jnp.maximum(h, 0.0)
    out[0] = h[:, :HD]
    out[1] = h[:, HD:]


def _dense_tc(agg, x, W_rel, b_rel, W_root):
    b2d = b_rel.reshape(1, D)
    grid = (NBLK,)
    return pl.pallas_call(
        _dense_body,
        grid=grid,
        in_specs=[
            pl.BlockSpec((NC, BLK, HD), lambda k: (0, k, 0)),
            pl.BlockSpec((BLK, D), lambda k: (k, 0)),
            pl.BlockSpec((D, D), lambda k: (0, 0)),
            pl.BlockSpec((D, D), lambda k: (0, 0)),
            pl.BlockSpec((1, D), lambda k: (0, 0)),
        ],
        out_specs=pl.BlockSpec((NC, BLK, HD), lambda k: (0, k, 0)),
        out_shape=jax.ShapeDtypeStruct((NC, N, HD), jnp.float32),
    )(agg, x, W_rel, W_root, b2d)


def _head_body(ab, hb, wr, wt, br, batchb, wl1, bl1, wl2, bl2,
               out, psum, cnt):
    k = pl.program_id(0)

    @pl.when(k == 0)
    def _init():
        psum[...] = jnp.zeros_like(psum)
        cnt[...] = jnp.zeros_like(cnt)

    s = jnp.concatenate([ab[0], ab[1]], axis=1)
    hin = jnp.concatenate([hb[0], hb[1]], axis=1)
    h = _dotT(s, wr[...]) + _dotT(hin, wt[...]) + br[...]
    h = jnp.maximum(h, 0.0)
    bvec = batchb[0, 0, :]
    onehot = (lax.broadcasted_iota(jnp.int32, (G, BLK), 0)
              == bvec[None, :]).astype(jnp.float32)
    psum[...] += jnp.dot(onehot, h, preferred_element_type=jnp.float32)
    cnt[...] += jnp.broadcast_to(
        jnp.sum(onehot, axis=1, keepdims=True), (G, D))

    @pl.when(k == NBLK - 1)
    def _fin():
        p = psum[...] / jnp.maximum(cnt[...], 1.0)
        z = jnp.maximum(_dotT(p, wl1[...]) + bl1[...], 0.0)
        out[...] = _dotT(z, wl2[...]) + bl2[...]


def _head_tc(agg, h1, W_rel, b_rel, W_root, batchr, W_lin1, b_lin1,
             W_lin2p, b_lin2p):
    b2d = b_rel.reshape(1, D)
    grid = (NBLK,)
    return pl.pallas_call(
        _head_body,
        grid=grid,
        in_specs=[
            pl.BlockSpec((NC, BLK, HD), lambda k: (0, k, 0)),
            pl.BlockSpec((NC, BLK, HD), lambda k: (0, k, 0)),
            pl.BlockSpec((D, D), lambda k: (0, 0)),
            pl.BlockSpec((D, D), lambda k: (0, 0)),
            pl.BlockSpec((1, D), lambda k: (0, 0)),
            pl.BlockSpec((1, 1, BLK), lambda k: (k, 0, 0)),
            pl.BlockSpec((16, D), lambda k: (0, 0)),
            pl.BlockSpec((1, 16), lambda k: (0, 0)),
            pl.BlockSpec((16, 16), lambda k: (0, 0)),
            pl.BlockSpec((1, 16), lambda k: (0, 0)),
        ],
        out_specs=pl.BlockSpec((G, 16), lambda k: (0, 0)),
        out_shape=jax.ShapeDtypeStruct((G, 16), jnp.float32),
        scratch_shapes=[
            pltpu.VMEM((G, D), jnp.float32),
            pltpu.VMEM((G, D), jnp.float32),
        ],
    )(agg, h1, W_rel, W_root, b2d, batchr,
      W_lin1, b_lin1.reshape(1, 16), W_lin2p, b_lin2p)


def kernel(x, edge_index, edge_attr, batch, W1_rel, b1_rel, W1_root,
           W2_rel, b2_rel, W2_root, W_lin1, b_lin1, W_lin2, b_lin2):
    src2 = edge_index[0].reshape(E // C, C)
    dst2 = edge_index[1].reshape(E // C, C)
    w2 = edge_attr.reshape(E // C, C)
    batchr = batch.reshape(NBLK, 1, BLK)
    W_lin2p = jnp.zeros((16, 16), jnp.float32).at[0].set(W_lin2[0])
    b_lin2p = jnp.zeros((1, 16), jnp.float32).at[0, 0].set(b_lin2[0])
    xsplit = x.reshape(N, NC, HD).swapaxes(0, 1)

    agg1 = _spmm_sc(xsplit, src2, dst2, w2)
    h1 = _dense_tc(agg1, x, W1_rel, b1_rel, W1_root)
    agg2 = _spmm_sc(h1, src2, dst2, w2)
    out16 = _head_tc(agg2, h1, W2_rel, b2_rel, W2_root, batchr,
                     W_lin1, b_lin1, W_lin2p, b_lin2p)
    return out16[:, 0:1]


# speedup vs baseline: 2.7629x; 1.1238x over previous
import jax
import jax.numpy as jnp
from jax import lax
from jax.experimental import pallas as pl
from jax.experimental.pallas import tpu as pltpu
from jax.experimental.pallas import tpu_sc as plsc

N = 10000
E = 320000
D = 128
G = 64

NC = 2
NS = 16
HD = D // NC
C = 100
EPT = E // NS
KCH = EPT // C
NBUF = 4
RPT = N // NS
ZR = 25
WPT = 632
WLAST = N - (NS - 1) * WPT

BLK = 400
NBLK = N // BLK


_SPLAT_DNUMS = lax.GatherDimensionNumbers(
    offset_dims=(), collapsed_slice_dims=(0,), start_index_map=(0,))


def _splat(vec16, lane):
    idx = jnp.full((16, 1), lane, jnp.int32)
    return lax.gather(vec16, idx, _SPLAT_DNUMS, (1,),
                      mode=lax.GatherScatterMode.PROMISE_IN_BOUNDS)


def _spmm_body(x2_hbm, src_hbm, dst_hbm, w_hbm, out_hbm,
               acc, sidx, didx, wbuf, ring, zbuf, gsem, ssem):
    cid = lax.axis_index("c")
    sid = lax.axis_index("s")
    xh = x2_hbm.at[cid]

    zero16 = jnp.zeros((16,), jnp.float32)
    for r in range(ZR):
        for j in range(HD // 16):
            zbuf[r, pl.ds(16 * j, 16)] = zero16
    for k in range(RPT // ZR):
        pltpu.sync_copy(zbuf, acc.at[pl.ds(sid * RPT + k * ZR, ZR)])

    pltpu.sync_copy(src_hbm.at[pl.ds(sid * KCH, KCH)], sidx)
    pltpu.sync_copy(dst_hbm.at[pl.ds(sid * KCH, KCH)], didx)
    pltpu.sync_copy(w_hbm.at[pl.ds(sid * KCH, KCH)], wbuf)
    plsc.subcore_barrier()

    for p in range(NBUF - 1):
        pltpu.async_copy(xh.at[sidx.at[p]], ring.at[p], gsem)

    def chunk(k, carry):
        b = lax.rem(k, NBUF)
        pltpu.make_async_copy(xh.at[sidx.at[k]], ring.at[b], gsem).wait()

        def scale_run(goff, lo):
            wv = wbuf[k, pl.ds(goff, 16)]
            for i16 in range(lo, 16):
                ws = _splat(wv, i16)
                e = goff + i16
                for j in range(HD // 16):
                    sl = pl.ds(16 * j, 16)
                    ring[b, e, sl] = ring[b, e, sl] * ws
        for g in range(C // 16):
            scale_run(g * 16, 0)
        if C % 16:
            scale_run(C - 16, 16 - C % 16)

        pltpu.async_copy(ring.at[b], acc.at[didx.at[k]], ssem, add=True)

        @pl.when(k >= 2)
        def _drain():
            pb = lax.rem(k - 2, NBUF)
            pltpu.make_async_copy(ring.at[pb], acc.at[didx.at[k - 2]],
                                  ssem).wait()

        @pl.when(k + NBUF - 1 < KCH)
        def _prefetch():
            nb = lax.rem(k + NBUF - 1, NBUF)
            pltpu.async_copy(xh.at[sidx.at[k + NBUF - 1]], ring.at[nb], gsem)
        return carry
    lax.fori_loop(0, KCH, chunk, 0)
    for t in (KCH - 2, KCH - 1):
        pltpu.make_async_copy(ring.at[lax.rem(t, NBUF)],
                              acc.at[didx.at[t]], ssem).wait()

    plsc.subcore_barrier()

    @pl.when(sid < NS - 1)
    def _wmain():
        pltpu.sync_copy(acc.at[pl.ds(sid * WPT, WPT)],
                        out_hbm.at[cid, pl.ds(sid * WPT, WPT)])

    @pl.when(sid == NS - 1)
    def _wtail():
        pltpu.sync_copy(acc.at[pl.ds((NS - 1) * WPT, WLAST)],
                        out_hbm.at[cid, pl.ds((NS - 1) * WPT, WLAST)])


def _spmm_sc(x2, src2, dst2, w2):
    mesh = plsc.VectorSubcoreMesh(core_axis_name="c", subcore_axis_name="s",
                                  num_cores=NC, num_subcores=NS)
    f = pl.kernel(
        _spmm_body,
        out_type=jax.ShapeDtypeStruct((NC, N, HD), jnp.float32),
        mesh=mesh,
        compiler_params=pltpu.CompilerParams(use_tc_tiling_on_sc=False,
                                             needs_layout_passes=False),
        scratch_types=[
            pltpu.VMEM_SHARED((N, HD), jnp.float32),
            pltpu.VMEM((KCH, C), jnp.int32),
            pltpu.VMEM((KCH, C), jnp.int32),
            pltpu.VMEM((KCH, C), jnp.float32),
            pltpu.VMEM((NBUF, C, HD), jnp.float32),
            pltpu.VMEM((ZR, HD), jnp.float32),
            pltpu.SemaphoreType.DMA,
            pltpu.SemaphoreType.DMA,
        ],
    )
    return f(x2, src2, dst2, w2)


def _dotT(a, b):
    return lax.dot_general(a, b, (((1,), (1,)), ((), ())),
                           preferred_element_type=jnp.float32)


def _root_body(xb, wt, br, out):
    out[...] = _dotT(xb[...], wt[...]) + br[...]


def _root_tc(x, W_root, b_rel):
    return pl.pallas_call(
        _root_body,
        grid=(NBLK,),
        in_specs=[
            pl.BlockSpec((BLK, D), lambda k: (k, 0)),
            pl.BlockSpec((D, D), lambda k: (0, 0)),
            pl.BlockSpec((1, D), lambda k: (0, 0)),
        ],
        out_specs=pl.BlockSpec((BLK, D), lambda k: (k, 0)),
        out_shape=jax.ShapeDtypeStruct((N, D), jnp.float32),
    )(x, W_root, b_rel.reshape(1, D))


def _rel_body(ab, rootb, wr, out):
    s = jnp.concatenate([ab[0], ab[1]], axis=1)
    h = jnp.maximum(_dotT(s, wr[...]) + rootb[...], 0.0)
    out[0] = h[:, :HD]
    out[1] = h[:, HD:]


def _rel_tc(agg, root, W_rel):
    return pl.pallas_call(
        _rel_body,
        grid=(NBLK,),
        in_specs=[
            pl.BlockSpec((NC, BLK, HD), lambda k: (0, k, 0)),
            pl.BlockSpec((BLK, D), lambda k: (k, 0)),
            pl.BlockSpec((D, D), lambda k: (0, 0)),
        ],
        out_specs=pl.BlockSpec((NC, BLK, HD), lambda k: (0, k, 0)),
        out_shape=jax.ShapeDtypeStruct((NC, N, HD), jnp.float32),
    )(agg, root, W_rel)


def _root2_body(hb, wt, br, out):
    hin = jnp.concatenate([hb[0], hb[1]], axis=1)
    out[...] = _dotT(hin, wt[...]) + br[...]


def _root2_tc(h1s, W_root, b_rel):
    return pl.pallas_call(
        _root2_body,
        grid=(NBLK,),
        in_specs=[
            pl.BlockSpec((NC, BLK, HD), lambda k: (0, k, 0)),
            pl.BlockSpec((D, D), lambda k: (0, 0)),
            pl.BlockSpec((1, D), lambda k: (0, 0)),
        ],
        out_specs=pl.BlockSpec((BLK, D), lambda k: (k, 0)),
        out_shape=jax.ShapeDtypeStruct((N, D), jnp.float32),
    )(h1s, W_root, b_rel.reshape(1, D))


def _head_body(ab, rootb, wr, batchb, wl1, bl1, wl2, bl2,
               out, psum, cnt):
    k = pl.program_id(0)

    @pl.when(k == 0)
    def _init():
        psum[...] = jnp.zeros_like(psum)
        cnt[...] = jnp.zeros_like(cnt)

    s = jnp.concatenate([ab[0], ab[1]], axis=1)
    h = jnp.maximum(_dotT(s, wr[...]) + rootb[...], 0.0)
    bvec = batchb[0, 0, :]
    onehot = (lax.broadcasted_iota(jnp.int32, (G, BLK), 0)
              == bvec[None, :]).astype(jnp.float32)
    psum[...] += jnp.dot(onehot, h, preferred_element_type=jnp.float32)
    cnt[...] += jnp.broadcast_to(
        jnp.sum(onehot, axis=1, keepdims=True), (G, D))

    @pl.when(k == NBLK - 1)
    def _fin():
        p = psum[...] / jnp.maximum(cnt[...], 1.0)
        z = jnp.maximum(_dotT(p, wl1[...]) + bl1[...], 0.0)
        out[...] = _dotT(z, wl2[...]) + bl2[...]


def _head_tc(agg, root2, W_rel, batchr, W_lin1, b_lin1, W_lin2p, b_lin2p):
    return pl.pallas_call(
        _head_body,
        grid=(NBLK,),
        in_specs=[
            pl.BlockSpec((NC, BLK, HD), lambda k: (0, k, 0)),
            pl.BlockSpec((BLK, D), lambda k: (k, 0)),
            pl.BlockSpec((D, D), lambda k: (0, 0)),
            pl.BlockSpec((1, 1, BLK), lambda k: (k, 0, 0)),
            pl.BlockSpec((16, D), lambda k: (0, 0)),
            pl.BlockSpec((1, 16), lambda k: (0, 0)),
            pl.BlockSpec((16, 16), lambda k: (0, 0)),
            pl.BlockSpec((1, 16), lambda k: (0, 0)),
        ],
        out_specs=pl.BlockSpec((G, 16), lambda k: (0, 0)),
        out_shape=jax.ShapeDtypeStruct((G, 16), jnp.float32),
        scratch_shapes=[
            pltpu.VMEM((G, D), jnp.float32),
            pltpu.VMEM((G, D), jnp.float32),
        ],
    )(agg, root2, W_rel, batchr,
      W_lin1, b_lin1.reshape(1, 16), W_lin2p, b_lin2p)


def kernel(x, edge_index, edge_attr, batch, W1_rel, b1_rel, W1_root,
           W2_rel, b2_rel, W2_root, W_lin1, b_lin1, W_lin2, b_lin2):
    src2 = edge_index[0].reshape(E // C, C)
    dst2 = edge_index[1].reshape(E // C, C)
    w2 = edge_attr.reshape(E // C, C)
    batchr = batch.reshape(NBLK, 1, BLK)
    W_lin2p = jnp.zeros((16, 16), jnp.float32).at[0].set(W_lin2[0])
    b_lin2p = jnp.zeros((1, 16), jnp.float32).at[0, 0].set(b_lin2[0])

    xsplit = x.reshape(N, NC, HD).swapaxes(0, 1)
    agg1 = _spmm_sc(xsplit, src2, dst2, w2)
    root1 = _root_tc(x, W1_root, b1_rel)
    h1s = _rel_tc(agg1, root1, W1_rel)
    agg2 = _spmm_sc(h1s, src2, dst2, w2)
    root2 = _root2_tc(h1s, W2_root, b2_rel)
    out16 = _head_tc(agg2, root2, W2_rel, batchr,
                     W_lin1, b_lin1, W_lin2p, b_lin2p)
    return out16[:, 0:1]
